# baseline (speedup 1.0000x reference)
import jax
import jax.numpy as jnp
from jax.experimental import pallas as pl
from jax.experimental.pallas import tpu as pltpu

N = 50000
D = 512
NUM_GRAPHS = 64
BLOCK_R = 5000


def _pool_kernel(seg_ref, h_ref, w_ref, out_ref, acc_ref, m_ref, z_ref):
    i = pl.program_id(0)
    nsteps = pl.num_programs(0)

    @pl.when(i == 0)
    def _init():
        acc_ref[...] = jnp.zeros_like(acc_ref)
        m_ref[0, 0] = -jnp.inf
        z_ref[0, 0] = 0.0

    hb = h_ref[...].astype(jnp.bfloat16)
    s8 = jax.lax.dot_general(
        hb, w_ref[...].astype(jnp.bfloat16), (((1,), (1,)), ((), ())),
        preferred_element_type=jnp.float32)
    s = s8[:, 0:1]

    m_old = m_ref[0, 0]
    m_new = jnp.maximum(m_old, jnp.max(s))
    alpha = jnp.exp(m_old - m_new)
    p = jnp.exp(s - m_new)

    m_ref[0, 0] = m_new
    z_ref[0, 0] = z_ref[0, 0] * alpha + jnp.sum(p)

    seg = seg_ref[0, 0, :].reshape(BLOCK_R, 1)
    gid = jax.lax.broadcasted_iota(jnp.int32, (BLOCK_R, NUM_GRAPHS), 1)
    onehot_p = jnp.where(gid == seg, p, 0.0).astype(jnp.bfloat16)

    contrib = jax.lax.dot_general(
        onehot_p, hb, (((0,), (0,)), ((), ())),
        preferred_element_type=jnp.float32)
    acc_ref[...] = acc_ref[...] * alpha + contrib

    @pl.when(i == nsteps - 1)
    def _finish():
        out_ref[...] = acc_ref[...] / z_ref[0, 0]


@jax.jit
def kernel(h, segment_ids, W, b):
    del b
    nsteps = N // BLOCK_R
    seg = segment_ids.astype(jnp.int32).reshape(nsteps, 1, BLOCK_R)
    w8 = jnp.broadcast_to(W, (8, D))
    return pl.pallas_call(
        _pool_kernel,
        grid=(nsteps,),
        in_specs=[
            pl.BlockSpec((1, 1, BLOCK_R), lambda i: (i, 0, 0)),
            pl.BlockSpec((BLOCK_R, D), lambda i: (i, 0)),
            pl.BlockSpec((8, D), lambda i: (0, 0)),
        ],
        out_specs=pl.BlockSpec((NUM_GRAPHS, D), lambda i: (0, 0)),
        out_shape=jax.ShapeDtypeStruct((NUM_GRAPHS, D), jnp.float32),
        scratch_shapes=[
            pltpu.VMEM((NUM_GRAPHS, D), jnp.float32),
            pltpu.SMEM((1, 1), jnp.float32),
            pltpu.SMEM((1, 1), jnp.float32),
        ],
    )(seg, h, w8)


# speedup vs baseline: 1.3709x; 1.3709x over previous
import jax
import jax.numpy as jnp
from jax.experimental import pallas as pl
from jax.experimental.pallas import tpu as pltpu

N = 50000
D = 512
NUM_GRAPHS = 64
BLOCK_R = 10000


def _pool_kernel(seg_ref, h_ref, w_ref, out_ref, acc_ref, m_ref, z_ref):
    i = pl.program_id(0)
    nsteps = pl.num_programs(0)

    @pl.when(i == 0)
    def _init():
        acc_ref[...] = jnp.zeros_like(acc_ref)
        m_ref[0, 0] = -jnp.inf
        z_ref[0, 0] = 0.0

    h = h_ref[...]
    s = jax.lax.dot_general(
        w_ref[...], h, (((1,), (1,)), ((), ())),
        preferred_element_type=jnp.float32)

    m_old = m_ref[0, 0]
    m_new = jnp.maximum(m_old, jnp.max(s))
    alpha = jnp.exp(m_old - m_new)
    p = jnp.exp(s - m_new)

    m_ref[0, 0] = m_new
    z_ref[0, 0] = z_ref[0, 0] * alpha + jnp.sum(p)

    seg = seg_ref[0, :, :]
    gid = jax.lax.broadcasted_iota(jnp.int32, (NUM_GRAPHS, BLOCK_R), 0)
    onehot_p = jnp.where(gid == seg, p, 0.0)

    contrib = jax.lax.dot_general(
        onehot_p, h, (((1,), (0,)), ((), ())),
        preferred_element_type=jnp.float32)
    acc_ref[...] = acc_ref[...] * alpha + contrib

    @pl.when(i == nsteps - 1)
    def _finish():
        out_ref[...] = acc_ref[...] / z_ref[0, 0]


@jax.jit
def kernel(h, segment_ids, W, b):
    del b
    nsteps = N // BLOCK_R
    seg = segment_ids.astype(jnp.int32).reshape(nsteps, 1, BLOCK_R)
    return pl.pallas_call(
        _pool_kernel,
        grid=(nsteps,),
        in_specs=[
            pl.BlockSpec((1, 1, BLOCK_R), lambda i: (i, 0, 0)),
            pl.BlockSpec((BLOCK_R, D), lambda i: (i, 0)),
            pl.BlockSpec((1, D), lambda i: (0, 0)),
        ],
        out_specs=pl.BlockSpec((NUM_GRAPHS, D), lambda i: (0, 0)),
        out_shape=jax.ShapeDtypeStruct((NUM_GRAPHS, D), jnp.float32),
        scratch_shapes=[
            pltpu.VMEM((NUM_GRAPHS, D), jnp.float32),
            pltpu.SMEM((1, 1), jnp.float32),
            pltpu.SMEM((1, 1), jnp.float32),
        ],
    )(seg, h, W)
